# baseline (speedup 1.0000x reference)
import jax
import jax.numpy as jnp
from jax import lax
from jax.experimental import pallas as pl
from jax.experimental.pallas import tpu as pltpu
from jax.experimental.pallas import tpu_sc as plsc

B = 1024
K = 5
SYN = 64
SEM = 32
LANES = 128
HV_PER_SUP = LANES // SEM
CAT_PER_SUP = LANES // SYN
NC = 2
NS = 16
NW = NC * NS
CAT_PW = B // NW
HV_TOTAL = 2 * B * K
HV_CHUNK = 80
HV_CHUNKS_PW = HV_TOTAL // (NW * HV_CHUNK)
HV_PW = HV_CHUNK * HV_CHUNKS_PW


def _sc_body(cat_ix_hbm, hv_ix_hbm, cat_sup_hbm, hvec_sup_hbm,
             cat_out_hbm, hv_out_hbm,
             cat_idx_v, hv_idx_v, cat_rows_v, hv_rows_v, sem):
    wid = lax.axis_index("s") * NC + lax.axis_index("c")
    pltpu.sync_copy(cat_ix_hbm.at[pl.ds(wid * CAT_PW, CAT_PW)], cat_idx_v)
    pltpu.sync_copy(hv_ix_hbm.at[pl.ds(wid * HV_CHUNKS_PW, HV_CHUNKS_PW)],
                    hv_idx_v)
    copies = [pltpu.async_copy(cat_sup_hbm.at[cat_idx_v], cat_rows_v, sem)]
    for j in range(HV_CHUNKS_PW):
        copies.append(pltpu.async_copy(
            hvec_sup_hbm.at[hv_idx_v.at[j]],
            hv_rows_v.at[pl.ds(j * HV_CHUNK, HV_CHUNK)], sem))
    for c in copies:
        c.wait()
    pltpu.sync_copy(cat_rows_v, cat_out_hbm.at[pl.ds(wid * CAT_PW, CAT_PW)])
    pltpu.sync_copy(hv_rows_v, hv_out_hbm.at[pl.ds(wid * HV_PW, HV_PW)])


@jax.jit
def _sc_gather(cat_sup_ix, hv_sup_ix, cat_sup, hvec_sup):
    mesh = plsc.VectorSubcoreMesh(core_axis_name="c", subcore_axis_name="s")
    return pl.kernel(
        _sc_body,
        mesh=mesh,
        out_type=(
            jax.ShapeDtypeStruct((B, LANES), jnp.float32),
            jax.ShapeDtypeStruct((HV_TOTAL, LANES), jnp.float32),
        ),
        scratch_types=[
            pltpu.VMEM((CAT_PW,), jnp.int32),
            pltpu.VMEM((HV_CHUNKS_PW, HV_CHUNK), jnp.int32),
            pltpu.VMEM((CAT_PW, LANES), jnp.float32),
            pltpu.VMEM((HV_PW, LANES), jnp.float32),
            pltpu.SemaphoreType.DMA,
        ],
    )(cat_sup_ix, hv_sup_ix, cat_sup, hvec_sup)


def _select_sub(rows, sub, width):
    out = (sub == 0).astype(jnp.float32) * rows[:, 0:width]
    for p in range(1, LANES // width):
        m = (sub == p).astype(jnp.float32)
        out = out + m * rows[:, p * width:(p + 1) * width]
    return out


def _dedup_weighted_sum(ix, hv, base):
    sub = jnp.bitwise_and(ix, HV_PER_SUP - 1)
    acc = _select_sub(hv[base], sub[:, 0:1], SEM)
    for j in range(1, K):
        dup = (ix[:, 0:1] == ix[:, j:j + 1]).astype(jnp.float32)
        for i in range(1, j):
            dup = jnp.maximum(
                dup, (ix[:, i:i + 1] == ix[:, j:j + 1]).astype(jnp.float32))
        acc = acc + (1.0 - dup) * _select_sub(hv[base + j], sub[:, j:j + 1],
                                              SEM)
    return acc


def _tc_body(cat_rows_ref, hv_rows_ref, cat_ix_ref, hvb_ix_ref, hvf_ix_ref,
             hvb_top_ref, hvf_top_ref, d_onehot_ref,
             w1_ref, b1_ref, w2_ref, b2_ref, out_ref):
    hv = hv_rows_ref[...]
    hvb_e = _dedup_weighted_sum(hvb_ix_ref[...], hv, 0) + hvb_top_ref[...]
    hvf_e = _dedup_weighted_sum(hvf_ix_ref[...], hv, K) + hvf_top_ref[...]
    cat_sub = jnp.bitwise_and(cat_ix_ref[...], CAT_PER_SUP - 1)
    cat_e = _select_sub(cat_rows_ref[...], cat_sub, SYN)
    x = jnp.concatenate([cat_e, hvb_e, hvf_e, d_onehot_ref[...]], axis=1)
    h = jnp.dot(x, w1_ref[...], preferred_element_type=jnp.float32)
    h = jnp.maximum(h + b1_ref[...], 0.0)
    o = jnp.dot(h, w2_ref[...], preferred_element_type=jnp.float32)
    o = o + b2_ref[...]
    m = jnp.max(o, axis=1, keepdims=True)
    s = o - m
    out_ref[...] = s - jnp.log(jnp.sum(jnp.exp(s), axis=1, keepdims=True))


@jax.jit
def _tc_mlp(cat_rows, hv_rows, cat_ix, hvb_ix, hvf_ix, hvb_top, hvf_top,
            d_onehot, w1, b1, w2, b2):
    out_dim = w2.shape[1]
    return pl.pallas_call(
        _tc_body,
        out_shape=jax.ShapeDtypeStruct((B, out_dim), jnp.float32),
    )(cat_rows, hv_rows, cat_ix, hvb_ix, hvf_ix, hvb_top, hvf_top,
      d_onehot, w1, b1, w2, b2)


def kernel(d_onehot, cat_b_ix, hvb_ix, hvf_ix, hvb_top, hvf_top, use_gpu,
           cat_emb, hvec_emb, fc1_w, fc1_b, fc2_w, fc2_b):
    cat_ix = cat_b_ix.astype(jnp.int32)
    hvb_i = hvb_ix.astype(jnp.int32)
    hvf_i = hvf_ix.astype(jnp.int32)
    hv_flat = jnp.concatenate([hvb_i.T.reshape(-1), hvf_i.T.reshape(-1)])
    hv_sup_ix = (hv_flat // HV_PER_SUP).reshape(HV_TOTAL // HV_CHUNK,
                                                HV_CHUNK)
    cat_sup_ix = cat_ix // CAT_PER_SUP
    cat_sup = cat_emb.reshape(-1, LANES)
    hvec_sup = hvec_emb.reshape(-1, LANES)
    cat_rows, hv_rows = _sc_gather(cat_sup_ix, hv_sup_ix, cat_sup, hvec_sup)
    return _tc_mlp(cat_rows, hv_rows.reshape(2 * K, B, LANES),
                   cat_ix.reshape(B, 1), hvb_i, hvf_i,
                   hvb_top, hvf_top, d_onehot,
                   fc1_w.T, fc1_b.reshape(1, -1),
                   fc2_w.T, fc2_b.reshape(1, -1))


# speedup vs baseline: 1.0602x; 1.0602x over previous
import jax
import jax.numpy as jnp
from jax import lax
from jax.experimental import pallas as pl
from jax.experimental.pallas import tpu as pltpu
from jax.experimental.pallas import tpu_sc as plsc

B = 1024
K = 5
SYN = 64
SEM = 32
NC = 2
NS = 16
NW = NC * NS
CAT_PW = B // NW
HV_TOTAL = 2 * B * K
HV_CHUNK = 80
HV_CHUNKS_PW = HV_TOTAL // (NW * HV_CHUNK)
HV_PW = HV_CHUNK * HV_CHUNKS_PW


def _sc_body(cat_ix_hbm, hv_ix_hbm, cat_emb_hbm, hvec_emb_hbm,
             cat_out_hbm, hv_out_hbm,
             cat_idx_v, hv_idx_v, cat_rows_v, hv_rows_v, sem):
    wid = lax.axis_index("s") * NC + lax.axis_index("c")
    pltpu.sync_copy(cat_ix_hbm.at[pl.ds(wid * CAT_PW, CAT_PW)], cat_idx_v)
    pltpu.sync_copy(hv_ix_hbm.at[pl.ds(wid * HV_CHUNKS_PW, HV_CHUNKS_PW)],
                    hv_idx_v)
    copies = [pltpu.async_copy(cat_emb_hbm.at[cat_idx_v], cat_rows_v, sem)]
    for j in range(HV_CHUNKS_PW):
        copies.append(pltpu.async_copy(
            hvec_emb_hbm.at[hv_idx_v.at[j]],
            hv_rows_v.at[pl.ds(j * HV_CHUNK, HV_CHUNK)], sem))
    for c in copies:
        c.wait()
    pltpu.sync_copy(cat_rows_v, cat_out_hbm.at[pl.ds(wid * CAT_PW, CAT_PW)])
    pltpu.sync_copy(hv_rows_v, hv_out_hbm.at[pl.ds(wid * HV_PW, HV_PW)])


@jax.jit
def _sc_gather(cat_ix, hv_ix, cat_emb, hvec_emb):
    mesh = plsc.VectorSubcoreMesh(core_axis_name="c", subcore_axis_name="s")
    return pl.kernel(
        _sc_body,
        mesh=mesh,
        out_type=(
            jax.ShapeDtypeStruct((B, SYN), jnp.float32),
            jax.ShapeDtypeStruct((HV_TOTAL, SEM), jnp.float32),
        ),
        scratch_types=[
            pltpu.VMEM((CAT_PW,), jnp.int32),
            pltpu.VMEM((HV_CHUNKS_PW, HV_CHUNK), jnp.int32),
            pltpu.VMEM((CAT_PW, SYN), jnp.float32),
            pltpu.VMEM((HV_PW, SEM), jnp.float32),
            pltpu.SemaphoreType.DMA,
        ],
        compiler_params=pltpu.CompilerParams(use_tc_tiling_on_sc=False),
    )(cat_ix, hv_ix, cat_emb, hvec_emb)


def _dedup_weighted_sum(ix, hv, base):
    acc = hv[base]
    for j in range(1, K):
        dup = (ix[:, 0:1] == ix[:, j:j + 1]).astype(jnp.float32)
        for i in range(1, j):
            dup = jnp.maximum(
                dup, (ix[:, i:i + 1] == ix[:, j:j + 1]).astype(jnp.float32))
        acc = acc + (1.0 - dup) * hv[base + j]
    return acc


def _matmul_t(x, w):
    return lax.dot_general(x, w, (((1,), (1,)), ((), ())),
                           preferred_element_type=jnp.float32)


def _tc_body(cat_rows_ref, hv_rows_ref, hvb_ix_ref, hvf_ix_ref,
             hvb_top_ref, hvf_top_ref, d_onehot_ref,
             w1_ref, b1_ref, w2_ref, b2_ref, out_ref):
    hv = hv_rows_ref[...]
    hvb_e = _dedup_weighted_sum(hvb_ix_ref[...], hv, 0) + hvb_top_ref[...]
    hvf_e = _dedup_weighted_sum(hvf_ix_ref[...], hv, K) + hvf_top_ref[...]
    x = jnp.concatenate(
        [cat_rows_ref[...], hvb_e, hvf_e, d_onehot_ref[...]], axis=1)
    h = jnp.maximum(_matmul_t(x, w1_ref[...]) + b1_ref[...], 0.0)
    o = _matmul_t(h, w2_ref[...]) + b2_ref[...]
    m = jnp.max(o, axis=1, keepdims=True)
    s = o - m
    out_ref[...] = s - jnp.log(jnp.sum(jnp.exp(s), axis=1, keepdims=True))


@jax.jit
def _tc_mlp(cat_rows, hv_rows, hvb_ix, hvf_ix, hvb_top, hvf_top, d_onehot,
            w1, b1, w2, b2):
    out_dim = w2.shape[0]
    return pl.pallas_call(
        _tc_body,
        out_shape=jax.ShapeDtypeStruct((B, out_dim), jnp.float32),
    )(cat_rows, hv_rows, hvb_ix, hvf_ix, hvb_top, hvf_top, d_onehot,
      w1, b1, w2, b2)


def kernel(d_onehot, cat_b_ix, hvb_ix, hvf_ix, hvb_top, hvf_top, use_gpu,
           cat_emb, hvec_emb, fc1_w, fc1_b, fc2_w, fc2_b):
    cat_ix = cat_b_ix.astype(jnp.int32)
    hvb_i = hvb_ix.astype(jnp.int32)
    hvf_i = hvf_ix.astype(jnp.int32)
    hv_flat = jnp.concatenate(
        [hvb_i.T.reshape(-1), hvf_i.T.reshape(-1)]
    ).reshape(HV_TOTAL // HV_CHUNK, HV_CHUNK)
    cat_rows, hv_rows = _sc_gather(cat_ix, hv_flat, cat_emb, hvec_emb)
    return _tc_mlp(cat_rows, hv_rows.reshape(2 * K, B, SEM),
                   hvb_i, hvf_i, hvb_top, hvf_top, d_onehot,
                   fc1_w, fc1_b.reshape(1, -1),
                   fc2_w, fc2_b.reshape(1, -1))
